# baseline (device time: 13724 ns/iter reference)
import jax
import jax.numpy as jnp
from jax import lax
from jax.experimental import pallas as pl
from jax.experimental.pallas import tpu as pltpu


def kernel(x, W, labels):
    T, D = x.shape
    _, V = W.shape
    VC = 512
    NC = V // VC
    labels2d = labels.reshape(1, T)

    def body(x_ref, w_ref, lab_ref, out_ref, acc, recv_buf, send_sem, recv_sem):
        j = pl.program_id(0)
        my_x = lax.axis_index("x")
        my_y = lax.axis_index("y")
        my_z = lax.axis_index("z")
        peer = (my_x, 1 - my_y, my_z)
        barrier = pltpu.get_barrier_semaphore()

        @pl.when(j == 0)
        def _():
            pl.semaphore_signal(
                barrier, inc=1, device_id=peer,
                device_id_type=pl.DeviceIdType.MESH,
            )

        xb = x_ref[:, :].astype(jnp.bfloat16)
        wb = w_ref[:, :].astype(jnp.bfloat16)
        logits_t = lax.dot_general(
            wb, xb,
            dimension_numbers=(((0,), (1,)), ((), ())),
            preferred_element_type=jnp.float32,
        )

        s_part = jnp.sum(jnp.exp(logits_t), axis=0)
        local_label = lab_ref[0, :] - my_y * V
        row_ids = j * VC + lax.broadcasted_iota(jnp.int32, (VC, T), 0)
        hit = row_ids == local_label[None, :]
        c_part = jnp.sum(jnp.where(hit, logits_t, 0.0), axis=0)

        @pl.when(j == 0)
        def _():
            acc[0, :] = s_part
            acc[1, :] = c_part

        @pl.when(j > 0)
        def _():
            acc[0, :] = acc[0, :] + s_part
            acc[1, :] = acc[1, :] + c_part

        @pl.when(j == NC - 1)
        def _():
            pl.semaphore_wait(barrier, 1)
            rdma = pltpu.make_async_remote_copy(
                src_ref=acc,
                dst_ref=recv_buf,
                send_sem=send_sem,
                recv_sem=recv_sem,
                device_id=peer,
                device_id_type=pl.DeviceIdType.MESH,
            )
            rdma.start()
            rdma.wait()
            s = acc[0, :] + recv_buf[0, :]
            c = acc[1, :] + recv_buf[1, :]
            out_ref[0, :] = jnp.log(s) - c

    out = pl.pallas_call(
        body,
        grid=(NC,),
        out_shape=jax.ShapeDtypeStruct((1, T), jnp.float32),
        in_specs=[
            pl.BlockSpec(memory_space=pltpu.VMEM),
            pl.BlockSpec((D, VC), lambda j: (0, j)),
            pl.BlockSpec(memory_space=pltpu.VMEM),
        ],
        out_specs=pl.BlockSpec((1, T), lambda j: (0, 0)),
        scratch_shapes=[
            pltpu.VMEM((8, T), jnp.float32),
            pltpu.VMEM((8, T), jnp.float32),
            pltpu.SemaphoreType.DMA,
            pltpu.SemaphoreType.DMA,
        ],
        compiler_params=pltpu.CompilerParams(collective_id=0),
    )(x, W, labels2d)
    return out.reshape(T)


# device time: 12104 ns/iter; 1.1338x vs baseline; 1.1338x over previous
import jax
import jax.numpy as jnp
from jax import lax
from jax.experimental import pallas as pl
from jax.experimental.pallas import tpu as pltpu

N_CHUNKS = 2


def kernel(x, W, labels):
    T, D = x.shape
    _, V = W.shape
    VC = V // N_CHUNKS
    labels2d = labels.reshape(1, T)

    def body(x_ref, w_ref, lab_ref, out_ref, acc, recv_buf,
             send_sems, recv_sems):
        my_x = lax.axis_index("x")
        my_y = lax.axis_index("y")
        my_z = lax.axis_index("z")
        peer = (my_x, 1 - my_y, my_z)

        barrier = pltpu.get_barrier_semaphore()
        pl.semaphore_signal(
            barrier, inc=1, device_id=peer,
            device_id_type=pl.DeviceIdType.MESH,
        )

        xb = x_ref[:, :].astype(jnp.bfloat16)
        local_label = lab_ref[0, :] - my_y * V

        rdmas = []
        for k in range(N_CHUNKS):
            wb = w_ref[:, pl.ds(k * VC, VC)].astype(jnp.bfloat16)
            logits_t = lax.dot_general(
                wb, xb,
                dimension_numbers=(((0,), (1,)), ((), ())),
                preferred_element_type=jnp.float32,
            )
            s = jnp.sum(jnp.exp(logits_t), axis=0)
            row_ids = k * VC + lax.broadcasted_iota(jnp.int32, (VC, T), 0)
            hit = row_ids == local_label[None, :]
            c = jnp.sum(jnp.where(hit, logits_t, 0.0), axis=0)
            acc[k, 0, :] = s
            acc[k, 1, :] = c
            if k == 0:
                pl.semaphore_wait(barrier, 1)
            rdma = pltpu.make_async_remote_copy(
                src_ref=acc.at[k],
                dst_ref=recv_buf.at[k],
                send_sem=send_sems.at[k],
                recv_sem=recv_sems.at[k],
                device_id=peer,
                device_id_type=pl.DeviceIdType.MESH,
            )
            rdma.start()
            rdmas.append(rdma)

        for r in rdmas:
            r.wait_recv()
        s_tot = acc[0, 0, :] + recv_buf[0, 0, :]
        c_tot = acc[0, 1, :] + recv_buf[0, 1, :]
        for k in range(1, N_CHUNKS):
            s_tot = s_tot + acc[k, 0, :] + recv_buf[k, 0, :]
            c_tot = c_tot + acc[k, 1, :] + recv_buf[k, 1, :]
        out_ref[0, :] = jnp.log(s_tot) - c_tot
        for r in rdmas:
            r.wait_send()

    out = pl.pallas_call(
        body,
        out_shape=jax.ShapeDtypeStruct((1, T), jnp.float32),
        in_specs=[
            pl.BlockSpec(memory_space=pltpu.VMEM),
            pl.BlockSpec(memory_space=pltpu.VMEM),
            pl.BlockSpec(memory_space=pltpu.VMEM),
        ],
        out_specs=pl.BlockSpec(memory_space=pltpu.VMEM),
        scratch_shapes=[
            pltpu.VMEM((N_CHUNKS, 8, T), jnp.float32),
            pltpu.VMEM((N_CHUNKS, 8, T), jnp.float32),
            pltpu.SemaphoreType.DMA((N_CHUNKS,)),
            pltpu.SemaphoreType.DMA((N_CHUNKS,)),
        ],
        compiler_params=pltpu.CompilerParams(collective_id=0),
    )(x, W, labels2d)
    return out.reshape(T)


# device time: 10906 ns/iter; 1.2584x vs baseline; 1.1098x over previous
import jax
import jax.numpy as jnp
from jax import lax
from jax.experimental import pallas as pl
from jax.experimental.pallas import tpu as pltpu


def kernel(x, W, labels):
    T, D = x.shape
    _, V = W.shape
    labels2d = labels.reshape(1, T)

    def body(x_ref, w_ref, lab_ref, out_ref, acc, recv_buf, send_sem, recv_sem):
        my_x = lax.axis_index("x")
        my_y = lax.axis_index("y")
        my_z = lax.axis_index("z")
        peer = (my_x, 1 - my_y, my_z)

        barrier = pltpu.get_barrier_semaphore()
        pl.semaphore_signal(
            barrier, inc=1, device_id=peer,
            device_id_type=pl.DeviceIdType.MESH,
        )

        xb = x_ref[:, :].astype(jnp.bfloat16)
        wb = w_ref[:, :].astype(jnp.bfloat16)
        logits_t = lax.dot_general(
            wb, xb,
            dimension_numbers=(((0,), (1,)), ((), ())),
            preferred_element_type=jnp.float32,
        )

        s = jnp.sum(jnp.exp(logits_t), axis=0)
        local_label = lab_ref[0, :] - my_y * V
        row_ids = lax.broadcasted_iota(jnp.int32, (V, T), 0)
        hit = row_ids == local_label[None, :]
        c = jnp.sum(jnp.where(hit, logits_t, 0.0), axis=0)

        acc[0, :] = s
        acc[1, :] = c

        pl.semaphore_wait(barrier, 1)
        rdma = pltpu.make_async_remote_copy(
            src_ref=acc,
            dst_ref=recv_buf,
            send_sem=send_sem,
            recv_sem=recv_sem,
            device_id=peer,
            device_id_type=pl.DeviceIdType.MESH,
        )
        rdma.start()
        rdma.wait()

        s_tot = acc[0, :] + recv_buf[0, :]
        c_tot = acc[1, :] + recv_buf[1, :]
        out_ref[0, :] = jnp.log(s_tot) - c_tot

    out = pl.pallas_call(
        body,
        out_shape=jax.ShapeDtypeStruct((1, T), jnp.float32),
        in_specs=[
            pl.BlockSpec(memory_space=pltpu.VMEM),
            pl.BlockSpec(memory_space=pltpu.VMEM),
            pl.BlockSpec(memory_space=pltpu.VMEM),
        ],
        out_specs=pl.BlockSpec(memory_space=pltpu.VMEM),
        scratch_shapes=[
            pltpu.VMEM((8, T), jnp.float32),
            pltpu.VMEM((8, T), jnp.float32),
            pltpu.SemaphoreType.DMA,
            pltpu.SemaphoreType.DMA,
        ],
        compiler_params=pltpu.CompilerParams(collective_id=0),
    )(x, W, labels2d)
    return out.reshape(T)
